# baseline (speedup 1.0000x reference)
import functools

import jax
import jax.numpy as jnp
from jax import lax
from jax.experimental import pallas as pl
from jax.experimental.pallas import tpu as pltpu
from jax.experimental.pallas import tpu_sc as plsc

_B = 1024
_F = 100
_D = 16
_V = 100
_TW = 64

_NC = 2
_NS = 16
_NW = _NC * _NS
_BPW = _B // _NW


def _sc_prep(x_hbm, ct_hbm, hc_hbm,
             idx_v, rows_v, acc_v, acc2_v, cc_v, sem):
    wid = lax.axis_index("s") * _NC + lax.axis_index("c")
    base = wid * _BPW

    def row_body(r, carry):
        pltpu.sync_copy(x_hbm.at[base + r], idx_v)
        pltpu.async_copy(ct_hbm.at[idx_v], rows_v, sem).wait()
        acc_v[...] = jnp.zeros((_D,), jnp.float32)
        acc2_v[...] = jnp.zeros((_D,), jnp.float32)

        def f_body(f, c):
            acc_v[...] = acc_v[...] + rows_v[f, pl.ds(0, _D)]
            acc2_v[...] = acc2_v[...] + rows_v[f, pl.ds(_D, _D)]
            return c

        lax.fori_loop(0, _F, f_body, 0, unroll=False)
        se = acc_v[...]
        cc_v[...] = 0.25 * (se * se) - 0.25 * acc2_v[...]
        pltpu.sync_copy(cc_v, hc_hbm.at[pl.ds((base + r) * _D, _D)])
        return carry

    lax.fori_loop(0, _BPW, row_body, 0, unroll=False)


def _writer_kernel(hc_ref, xt_ref, wcol_ref, out_ref):
    lin = jnp.sum(wcol_ref[...] * xt_ref[...], axis=0, keepdims=True)
    t = hc_ref[...] + 0.5 * lin
    out_ref[...] = 0.5 * jnp.tanh(t) + 0.5


def kernel(x, emb_table, linear_weights):
    xt = x.astype(jnp.float32).T
    wcol = linear_weights.reshape(_F, 1)
    ct = jnp.concatenate(
        [emb_table, emb_table * emb_table,
         jnp.zeros((_V, 128 - 2 * _D), jnp.float32)], axis=1)

    mesh = plsc.VectorSubcoreMesh(core_axis_name="c", subcore_axis_name="s")
    sc_fn = functools.partial(
        pl.kernel,
        mesh=mesh,
        out_type=jax.ShapeDtypeStruct((_B * _D,), jnp.float32),
        scratch_types=[
            pltpu.VMEM((_F,), jnp.int32),
            pltpu.VMEM((_F, 128), jnp.float32),
            pltpu.VMEM((_D,), jnp.float32),
            pltpu.VMEM((_D,), jnp.float32),
            pltpu.VMEM((_D,), jnp.float32),
            pltpu.SemaphoreType.DMA,
        ],
    )(_sc_prep)
    half_cross = sc_fn(x, ct).reshape(_B * _D, 1)

    out2 = pl.pallas_call(
        _writer_kernel,
        grid=(_B // _TW,),
        in_specs=[
            pl.BlockSpec((_TW * _D, 1), lambda i: (i, 0)),
            pl.BlockSpec((_F, _B), lambda i: (0, 0)),
            pl.BlockSpec((_F, 1), lambda i: (0, 0)),
        ],
        out_specs=pl.BlockSpec((_TW * _D, _B), lambda i: (i, 0)),
        out_shape=jax.ShapeDtypeStruct((_B * _D, _B), jnp.float32),
    )(half_cross, xt, wcol)

    return out2.reshape(_B, _D, _B).transpose(0, 2, 1)


# speedup vs baseline: 3.2934x; 3.2934x over previous
import jax
import jax.numpy as jnp
from jax.experimental import pallas as pl
from jax.experimental.pallas import tpu as pltpu

_B = 1024
_F = 100
_D = 16
_V = 100
_TI = 64
_TR = _TI * _D


def _fm_kernel(x_ref, xt_ref, wcol_ref, et_ref, et2_ref, out_ref):
    xb = x_ref[...]
    iota = jax.lax.broadcasted_iota(jnp.int32, (1, 1, _V), 2)
    eq = (xb[:, :, None] == iota).astype(jnp.int32)
    cmat = jnp.sum(eq, axis=1).astype(jnp.float32)
    cexp = jnp.broadcast_to(cmat[:, None, :], (_TI, _D, _V)).reshape(_TR, _V)
    eg = jnp.broadcast_to(et_ref[...][None, :, :], (_TI, _D, _V)).reshape(_TR, _V)
    eg2 = jnp.broadcast_to(et2_ref[...][None, :, :], (_TI, _D, _V)).reshape(_TR, _V)
    se = jnp.sum(cexp * eg, axis=1, keepdims=True)
    se2 = jnp.sum(cexp * eg2, axis=1, keepdims=True)
    half_cross = 0.25 * (se * se) - 0.25 * se2
    lin_row = jnp.sum(wcol_ref[...] * xt_ref[...], axis=0, keepdims=True)
    half_lin = 0.5 * lin_row
    t = half_cross + half_lin
    out_ref[...] = 0.5 * jnp.tanh(t) + 0.5


def kernel(x, emb_table, linear_weights):
    n_i = _B // _TI
    xt = x.astype(jnp.float32).T
    wcol = linear_weights.reshape(_F, 1)
    et = emb_table.T
    et2 = et * et

    out2 = pl.pallas_call(
        _fm_kernel,
        grid=(n_i,),
        in_specs=[
            pl.BlockSpec((_TI, _F), lambda i: (i, 0)),
            pl.BlockSpec((_F, _B), lambda i: (0, 0)),
            pl.BlockSpec((_F, 1), lambda i: (0, 0)),
            pl.BlockSpec((_D, _V), lambda i: (0, 0)),
            pl.BlockSpec((_D, _V), lambda i: (0, 0)),
        ],
        out_specs=pl.BlockSpec((_TR, _B), lambda i: (i, 0)),
        out_shape=jax.ShapeDtypeStruct((_B * _D, _B), jnp.float32),
        compiler_params=pltpu.CompilerParams(
            dimension_semantics=("parallel",),
        ),
    )(x, xt, wcol, et, et2)

    return out2.reshape(_B, _D, _B).transpose(0, 2, 1)
